# baseline (speedup 1.0000x reference)
import functools

import jax
import jax.numpy as jnp
from jax import lax
from jax.experimental import pallas as pl
from jax.experimental.pallas import tpu as pltpu
from jax.experimental.pallas import tpu_sc as plsc

_INFO = plsc.get_sparse_core_info()
_NC = _INFO.num_cores
_NS = _INFO.num_subcores
_NW = _NC * _NS
_CHUNK = 128


def _make_gather(num_rows: int, d: int):
    b_per_w = num_rows // _NW
    n_chunks = b_per_w // _CHUNK
    mesh = plsc.VectorSubcoreMesh(core_axis_name="c", subcore_axis_name="s")

    @functools.partial(
        pl.kernel,
        out_type=jax.ShapeDtypeStruct((_NW, b_per_w, d), jnp.float32),
        mesh=mesh,
        scratch_types=[
            pltpu.VMEM((n_chunks, _CHUNK), jnp.int32),
            pltpu.VMEM((b_per_w, d), jnp.float32),
            pltpu.SemaphoreType.DMA,
        ],
        compiler_params=pltpu.CompilerParams(use_tc_tiling_on_sc=False),
    )
    def gather_kernel(idx_hbm, table_hbm, out_hbm, idx_v, rows_v, sem):
        wid = lax.axis_index("s") * _NC + lax.axis_index("c")
        pltpu.sync_copy(idx_hbm.at[wid], idx_v)
        copies = []
        for k in range(n_chunks):
            copies.append(
                pltpu.async_copy(
                    table_hbm.at[idx_v.at[k]],
                    rows_v.at[pl.ds(k * _CHUNK, _CHUNK)],
                    sem,
                )
            )
        for c in copies:
            c.wait()
        pltpu.sync_copy(rows_v, out_hbm.at[wid])

    return gather_kernel


def kernel(text_tokens, predictions, tag_vocab):
    L = text_tokens.shape[0]
    d = tag_vocab.shape[1]
    slice_tags = predictions[0, -L:]
    idx = jnp.stack(
        [text_tokens.astype(jnp.int32), slice_tags.astype(jnp.int32)], axis=1
    ).reshape(_NW, (2 * L) // (_NW * _CHUNK), _CHUNK)
    rows = _make_gather(2 * L, d)(idx, tag_vocab)
    return rows.reshape(L, 2, d)


# speedup vs baseline: 2.6577x; 2.6577x over previous
import functools

import jax
import jax.numpy as jnp
from jax import lax
from jax.experimental import pallas as pl
from jax.experimental.pallas import tpu as pltpu
from jax.experimental.pallas import tpu_sc as plsc

_INFO = plsc.get_sparse_core_info()
_NC = _INFO.num_cores
_NS = _INFO.num_subcores
_NW = _NC * _NS
_LANES = _INFO.num_lanes
_UNROLL = 8


def _make_gather(d: int, vocab: int, n_idx: int):
    rows_per_w = d // _NW
    mesh = plsc.VectorSubcoreMesh(core_axis_name="c", subcore_axis_name="s")

    @functools.partial(
        pl.kernel,
        out_type=jax.ShapeDtypeStruct((2, d, n_idx), jnp.float32),
        mesh=mesh,
        scratch_types=[
            pltpu.VMEM((vocab,), jnp.float32),
            pltpu.VMEM((n_idx,), jnp.int32),
            pltpu.VMEM((n_idx,), jnp.int32),
            pltpu.VMEM((n_idx,), jnp.float32),
        ],
        compiler_params=pltpu.CompilerParams(needs_layout_passes=False),
    )
    def gather_kernel(idx_text_hbm, idx_tags_hbm, table_t_hbm, out_hbm,
                      row_v, idx_text_v, idx_tags_v, out_v):
        wid = lax.axis_index("s") * _NC + lax.axis_index("c")
        pltpu.sync_copy(idx_text_hbm, idx_text_v)
        pltpu.sync_copy(idx_tags_hbm, idx_tags_v)

        n_groups = n_idx // (_LANES * _UNROLL)

        for r in range(rows_per_w):
            comp = wid * rows_per_w + r
            pltpu.sync_copy(table_t_hbm.at[comp], row_v)
            for t, idx_v in ((0, idx_text_v), (1, idx_tags_v)):

                def body(g, _, idx_v=idx_v):
                    for j in range(_UNROLL):
                        off = (g * _UNROLL + j) * _LANES
                        iv = idx_v[pl.ds(off, _LANES)]
                        out_v[pl.ds(off, _LANES)] = plsc.load_gather(
                            row_v, [iv])
                    return 0

                lax.fori_loop(0, n_groups, body, 0)
                pltpu.sync_copy(out_v, out_hbm.at[t, comp])

    return gather_kernel


def kernel(text_tokens, predictions, tag_vocab):
    L = text_tokens.shape[0]
    vocab, d = tag_vocab.shape
    slice_tags = predictions[0, -L:]
    out_t = _make_gather(d, vocab, L)(
        text_tokens.astype(jnp.int32),
        slice_tags.astype(jnp.int32),
        tag_vocab.T,
    )
    return jnp.transpose(out_t, (2, 0, 1))
